# baseline (speedup 1.0000x reference)
import functools

import jax
import jax.numpy as jnp
from jax import lax
from jax.experimental import pallas as pl
from jax.experimental.pallas import tpu as pltpu
from jax.experimental.pallas import tpu_sc as plsc

_N = 10000
_JB = 200
_RB = 1000
_NW = _N // 8
_NWP = 1280


def _tf_body(x_ref, w_ref, al_ref, ar_ref, f_ref, el_ref, er_ref, *rest,
             H, D, tables):
    f = jnp.dot(x_ref[...], w_ref[...], preferred_element_type=jnp.float32)
    el = jnp.dot(f, al_ref[...], preferred_element_type=jnp.float32)
    er = jnp.dot(f, ar_ref[...], preferred_element_type=jnp.float32)
    el_ref[...] = el
    er_ref[...] = er
    ones = jnp.ones((f.shape[0], 1), dtype=jnp.bfloat16)
    fb = f.astype(jnp.bfloat16)
    pieces = []
    for h in range(H):
        pieces.append(fb[:, h * D:(h + 1) * D])
        pieces.append(ones)
    f_ref[...] = jnp.concatenate(pieces, axis=1)
    if tables:
        t_ref = rest[0]
        e0 = el[:, 0:1]
        r0 = er[:, 0:1]
        t_ref[...] = jnp.concatenate([
            jnp.exp(e0), jnp.exp(0.2 * e0),
            f[:, 0:1], f[:, 1:2],
            jnp.exp(r0), jnp.exp(0.2 * r0),
            jnp.zeros((f.shape[0], 2), jnp.float32),
        ], axis=1)


def _transform(h, W, AL, AR, H, D, tables=False):
    n = h.shape[0]
    kin = h.shape[1]
    kout = W.shape[1]
    hh = AL.shape[1]
    body = functools.partial(_tf_body, H=H, D=D, tables=tables)
    out_specs = [
        pl.BlockSpec((_RB, H * (D + 1)), lambda i: (i, 0)),
        pl.BlockSpec((_RB, hh), lambda i: (i, 0)),
        pl.BlockSpec((_RB, hh), lambda i: (i, 0)),
    ]
    out_shape = [
        jax.ShapeDtypeStruct((n, H * (D + 1)), jnp.bfloat16),
        jax.ShapeDtypeStruct((n, hh), jnp.float32),
        jax.ShapeDtypeStruct((n, hh), jnp.float32),
    ]
    if tables:
        out_specs.append(pl.BlockSpec((_RB, 8), lambda i: (i, 0)))
        out_shape.append(jax.ShapeDtypeStruct((n, 8), jnp.float32))
    return pl.pallas_call(
        body,
        grid=(n // _RB,),
        in_specs=[
            pl.BlockSpec((_RB, kin), lambda i: (i, 0)),
            pl.BlockSpec((kin, kout), lambda i: (0, 0)),
            pl.BlockSpec((kout, hh), lambda i: (0, 0)),
            pl.BlockSpec((kout, hh), lambda i: (0, 0)),
        ],
        out_specs=out_specs,
        out_shape=out_shape,
    )(h, W, AL, AR)


def _mp1_body(adj_ref, elT_ref, er_ref, feat_ref, bias_ref, m8_ref,
              out_ref, p_ref, *, H, D):
    adjb = adj_ref[...].astype(jnp.bfloat16)
    p_ref[0, :, :] = jnp.dot(m8_ref[...], adjb, preferred_element_type=jnp.float32)
    elT = elT_ref[...]
    er = er_ref[...]
    A = jnp.exp(elT).astype(jnp.bfloat16)
    C = jnp.exp(0.2 * elT).astype(jnp.bfloat16)
    B = jnp.exp(er).astype(jnp.bfloat16)
    Dd = jnp.exp(0.2 * er).astype(jnp.bfloat16)
    fbf = feat_ref[...]
    nums = []
    for h in range(H):
        a = A[h:h + 1, :]
        c = C[h:h + 1, :]
        b = B[:, h:h + 1]
        d = Dd[:, h:h + 1]
        w = adjb * jnp.maximum(b * a, d * c)
        nd = jnp.dot(w, fbf[:, h * (D + 1):(h + 1) * (D + 1)],
                     preferred_element_type=jnp.float32)
        den = nd[:, D:D + 1]
        safe = jnp.where(den > 0, den, 1.0)
        nums.append(nd[:, 0:D] / safe)
    res = jnp.concatenate(nums, axis=1) + bias_ref[...]
    out_ref[...] = jnp.where(res > 0, res,
                             jnp.exp(jnp.minimum(res, 0.0)) - 1.0)


def _message_pass1(adj, elT, er, feat, bias, m8, H, D):
    n = adj.shape[0]
    hd = feat.shape[1]
    wb = _JB // 8
    body = functools.partial(_mp1_body, H=H, D=D)
    return pl.pallas_call(
        body,
        grid=(n // _JB,),
        in_specs=[
            pl.BlockSpec((_JB, n), lambda j: (j, 0)),
            pl.BlockSpec((8, n), lambda j: (0, 0)),
            pl.BlockSpec((_JB, 8), lambda j: (j, 0)),
            pl.BlockSpec((n, hd), lambda j: (0, 0)),
            pl.BlockSpec((1, H * D), lambda j: (0, 0)),
            pl.BlockSpec((wb, _JB), lambda j: (0, 0)),
        ],
        out_specs=[
            pl.BlockSpec((_JB, H * D), lambda j: (j, 0)),
            pl.BlockSpec((1, wb, n), lambda j: (j, 0, 0)),
        ],
        out_shape=[
            jax.ShapeDtypeStruct((n, H * D), jnp.float32),
            jax.ShapeDtypeStruct((n // _JB, wb, n), jnp.float32),
        ],
        compiler_params=pltpu.CompilerParams(
            dimension_semantics=("arbitrary",),
        ),
    )(adj, elT, er, feat, bias, m8)


def _sc_layer2(pt, tbl, b2t, d2t):
    info = plsc.get_sparse_core_info()
    nc = info.num_cores
    nwk = nc * info.num_subcores
    mesh = plsc.VectorSubcoreMesh(core_axis_name="c", subcore_axis_name="s")
    nstrips = _N // 8
    npad = _NWP * 8

    @functools.partial(
        pl.kernel, mesh=mesh,
        compiler_params=pltpu.CompilerParams(needs_layout_passes=False),
        out_type=jax.ShapeDtypeStruct((_N, 48), jnp.float32),
        scratch_types=[
            pltpu.VMEM((4 * _NWP * 8,), jnp.float32),
            pltpu.VMEM((_NWP * 8,), jnp.float32),
            pltpu.VMEM((_NWP * 8,), jnp.float32),
            pltpu.VMEM((8, _NWP), jnp.float32),
            pltpu.VMEM((8, _NWP), jnp.float32),
            pltpu.VMEM((8, 48), jnp.float32),
            pltpu.SemaphoreType.DMA,
        ],
    )
    def k(pt_hbm, tbl_hbm, b2_hbm, d2_hbm, out_hbm,
          tblv, b2v, d2v, s0, s1, stage, sem):
        wid = lax.axis_index("s") * nc + lax.axis_index("c")
        pltpu.sync_copy(tbl_hbm, tblv)
        pltpu.sync_copy(b2_hbm, b2v)
        pltpu.sync_copy(d2_hbm, d2v)
        iot = lax.iota(jnp.int32, 16)
        zero = jnp.zeros((16,), jnp.float32)
        nt = (nstrips - 1 - wid) // nwk + 1

        def start(t, buf):
            g = wid + nwk * t
            pltpu.async_copy(pt_hbm.at[pl.ds(g * 8, 8), :], buf, sem)

        def wait(buf):
            pltpu.make_async_copy(pt_hbm.at[pl.ds(0, 8), :], buf, sem).wait()

        def process_strip(t, buf):
            g = wid + nwk * t
            dbase = g * 8
            for r in range(8):
                dst = dbase + r
                dvec = iot * 0 + dst
                bs = plsc.load_gather(b2v, [dvec])
                dsv = plsc.load_gather(d2v, [dvec])

                def scan_body(kk, acc):
                    wv = buf[r, pl.ds(kk * 16, 16)]

                    def dec(a):
                        wi = wv.astype(jnp.int32)
                        idx0 = kk * 128 + iot * 8
                        for s in range(8):
                            m = ((wi >> s) & 1) == 1

                            def hit(aa, m=m, s=s):
                                bn0, bn1, bd = aa
                                idxs = idx0 + s
                                av = plsc.load_gather(tblv, [idxs], mask=m)
                                cv = plsc.load_gather(tblv, [idxs + npad],
                                                      mask=m)
                                f0v = plsc.load_gather(tblv, [idxs + 2 * npad],
                                                       mask=m)
                                f1v = plsc.load_gather(tblv, [idxs + 3 * npad],
                                                       mask=m)
                                w = jnp.maximum(av * bs, cv * dsv)
                                w = jnp.where(m, w, 0.0)
                                return (bn0 + w * f0v, bn1 + w * f1v, bd + w)

                            a = lax.cond(jnp.any(m), hit, lambda aa: aa, a)
                        return a

                    return lax.cond(jnp.any(wv != 0.0), dec, lambda a: a, acc)

                an0, an1, ad = lax.fori_loop(0, _NWP // 16, scan_body,
                                             (zero, zero, zero))
                stage[r, pl.ds(0, 16)] = an0
                stage[r, pl.ds(16, 16)] = an1
                stage[r, pl.ds(32, 16)] = ad
            pltpu.sync_copy(stage, out_hbm.at[pl.ds(dbase, 8), :])

        @pl.when(nt > 0)
        def _prime():
            start(0, s0)

        def pair_body(p, carry):
            for b, buf, obuf in ((0, s0, s1), (1, s1, s0)):
                t = 2 * p + b

                @pl.when(t < nt)
                def _do():
                    @pl.when(t + 1 < nt)
                    def _next():
                        start(t + 1, obuf)
                    wait(buf)
                    process_strip(t, buf)
            return carry

        lax.fori_loop(0, (nt + 1) // 2, pair_body, jnp.int32(0))

    return k(pt, tbl, b2t, d2t)


def _epi_body(stg_ref, b_ref, out_ref):
    stg = stg_ref[...]
    n0 = jnp.sum(stg[:, 0:16], axis=1, keepdims=True)
    n1 = jnp.sum(stg[:, 16:32], axis=1, keepdims=True)
    den = jnp.sum(stg[:, 32:48], axis=1, keepdims=True)
    safe = jnp.where(den > 0, den, 1.0)
    lg = jnp.concatenate([n0, n1], axis=1) / safe + b_ref[...]
    m = jnp.max(lg, axis=1, keepdims=True)
    lse = m + jnp.log(jnp.sum(jnp.exp(lg - m), axis=1, keepdims=True))
    out_ref[...] = lg - lse


def _epilogue(stg, b2r):
    n = stg.shape[0]
    return pl.pallas_call(
        _epi_body,
        grid=(n // _RB,),
        in_specs=[
            pl.BlockSpec((_RB, 48), lambda i: (i, 0)),
            pl.BlockSpec((1, 2), lambda i: (0, 0)),
        ],
        out_specs=pl.BlockSpec((_RB, 2), lambda i: (i, 0)),
        out_shape=jax.ShapeDtypeStruct((n, 2), jnp.float32),
    )(stg, b2r)


def kernel(x, adj, W1, al1, ar1, b1, W2, al2, ar2, b2):
    eye = jnp.eye(8, dtype=jnp.float32)
    AL1 = (al1.reshape(8, 16)[:, :, None] * eye[:, None, :]).reshape(128, 8)
    AR1 = (ar1.reshape(8, 16)[:, :, None] * eye[:, None, :]).reshape(128, 8)
    AL2 = jnp.pad(al2.reshape(2, 1), ((0, 0), (0, 7)))
    AR2 = jnp.pad(ar2.reshape(2, 1), ((0, 0), (0, 7)))
    b1r = b1.reshape(1, 128)
    b2r = b2.reshape(1, 2)
    wb = _JB // 8
    rows = jnp.arange(_JB)
    m8 = jnp.where(rows[None, :] // 8 == jnp.arange(wb)[:, None],
                   2.0 ** (rows[None, :] % 8), 0.0).astype(jnp.bfloat16)

    f1, el1, er1 = _transform(x, W1, AL1, AR1, H=8, D=16)
    h1, P = _message_pass1(adj, el1.T, er1, f1, b1r, m8, H=8, D=16)
    P = P.reshape(_NW, _N)
    pt = jnp.pad(P.T, ((0, 0), (0, _NWP - _NW)))
    _, el2, er2, t8 = _transform(h1, W2, AL2, AR2, H=1, D=2, tables=True)
    zpad = jnp.zeros((_NWP * 8 - _N,), jnp.float32)
    tbl = jnp.concatenate([t8[:, 0], zpad, t8[:, 1], zpad,
                           t8[:, 2], zpad, t8[:, 3], zpad])
    stg = _sc_layer2(pt, tbl,
                     jnp.concatenate([t8[:, 4], zpad]),
                     jnp.concatenate([t8[:, 5], zpad]))
    return _epilogue(stg, b2r)


# speedup vs baseline: 1.1000x; 1.1000x over previous
import functools

import jax
import jax.numpy as jnp
from jax import lax
from jax.experimental import pallas as pl
from jax.experimental.pallas import tpu as pltpu
from jax.experimental.pallas import tpu_sc as plsc

_N = 10000
_JB = 200
_RB = 1000
_NW = _N // 8
_NWP = 1280


def _tf_body(x_ref, w_ref, al_ref, ar_ref, f_ref, el_ref, er_ref, *rest,
             H, D, tables):
    f = jnp.dot(x_ref[...], w_ref[...], preferred_element_type=jnp.float32)
    el = jnp.dot(f, al_ref[...], preferred_element_type=jnp.float32)
    er = jnp.dot(f, ar_ref[...], preferred_element_type=jnp.float32)
    el_ref[...] = el
    er_ref[...] = er
    ones = jnp.ones((f.shape[0], 1), dtype=jnp.bfloat16)
    fb = f.astype(jnp.bfloat16)
    pieces = []
    for h in range(H):
        pieces.append(fb[:, h * D:(h + 1) * D])
        pieces.append(ones)
    f_ref[...] = jnp.concatenate(pieces, axis=1)
    if tables:
        t_ref = rest[0]
        e0 = el[:, 0:1]
        r0 = er[:, 0:1]
        t_ref[...] = jnp.concatenate([
            jnp.exp(e0), jnp.exp(0.2 * e0),
            f[:, 0:1], f[:, 1:2],
            jnp.exp(r0), jnp.exp(0.2 * r0),
            jnp.zeros((f.shape[0], 2), jnp.float32),
        ], axis=1)


def _transform(h, W, AL, AR, H, D, tables=False):
    n = h.shape[0]
    kin = h.shape[1]
    kout = W.shape[1]
    hh = AL.shape[1]
    body = functools.partial(_tf_body, H=H, D=D, tables=tables)
    out_specs = [
        pl.BlockSpec((_RB, H * (D + 1)), lambda i: (i, 0)),
        pl.BlockSpec((_RB, hh), lambda i: (i, 0)),
        pl.BlockSpec((_RB, hh), lambda i: (i, 0)),
    ]
    out_shape = [
        jax.ShapeDtypeStruct((n, H * (D + 1)), jnp.bfloat16),
        jax.ShapeDtypeStruct((n, hh), jnp.float32),
        jax.ShapeDtypeStruct((n, hh), jnp.float32),
    ]
    if tables:
        out_specs.append(pl.BlockSpec((_RB, 8), lambda i: (i, 0)))
        out_shape.append(jax.ShapeDtypeStruct((n, 8), jnp.float32))
    return pl.pallas_call(
        body,
        grid=(n // _RB,),
        in_specs=[
            pl.BlockSpec((_RB, kin), lambda i: (i, 0)),
            pl.BlockSpec((kin, kout), lambda i: (0, 0)),
            pl.BlockSpec((kout, hh), lambda i: (0, 0)),
            pl.BlockSpec((kout, hh), lambda i: (0, 0)),
        ],
        out_specs=out_specs,
        out_shape=out_shape,
    )(h, W, AL, AR)


def _mp1_body(adj_ref, elT_ref, er_ref, feat_ref, bias_ref, m8_ref,
              out_ref, p_ref, *, H, D):
    adjb = adj_ref[...].astype(jnp.bfloat16)
    p_ref[0, :, :] = jnp.dot(m8_ref[...], adjb, preferred_element_type=jnp.float32)
    elT = elT_ref[...]
    er = er_ref[...]
    A = jnp.exp(elT).astype(jnp.bfloat16)
    C = jnp.exp(0.2 * elT).astype(jnp.bfloat16)
    B = jnp.exp(er).astype(jnp.bfloat16)
    Dd = jnp.exp(0.2 * er).astype(jnp.bfloat16)
    fbf = feat_ref[...]
    nums = []
    for h in range(H):
        a = A[h:h + 1, :]
        c = C[h:h + 1, :]
        b = B[:, h:h + 1]
        d = Dd[:, h:h + 1]
        w = adjb * jnp.maximum(b * a, d * c)
        nd = jnp.dot(w, fbf[:, h * (D + 1):(h + 1) * (D + 1)],
                     preferred_element_type=jnp.float32)
        den = nd[:, D:D + 1]
        safe = jnp.where(den > 0, den, 1.0)
        nums.append(nd[:, 0:D] / safe)
    res = jnp.concatenate(nums, axis=1) + bias_ref[...]
    out_ref[...] = jnp.where(res > 0, res,
                             jnp.exp(jnp.minimum(res, 0.0)) - 1.0)


def _message_pass1(adj, elT, er, feat, bias, m8, H, D):
    n = adj.shape[0]
    hd = feat.shape[1]
    wb = _JB // 8
    body = functools.partial(_mp1_body, H=H, D=D)
    return pl.pallas_call(
        body,
        grid=(n // _JB,),
        in_specs=[
            pl.BlockSpec((_JB, n), lambda j: (j, 0)),
            pl.BlockSpec((8, n), lambda j: (0, 0)),
            pl.BlockSpec((_JB, 8), lambda j: (j, 0)),
            pl.BlockSpec((n, hd), lambda j: (0, 0)),
            pl.BlockSpec((1, H * D), lambda j: (0, 0)),
            pl.BlockSpec((wb, _JB), lambda j: (0, 0)),
        ],
        out_specs=[
            pl.BlockSpec((_JB, H * D), lambda j: (j, 0)),
            pl.BlockSpec((1, wb, n), lambda j: (j, 0, 0)),
        ],
        out_shape=[
            jax.ShapeDtypeStruct((n, H * D), jnp.float32),
            jax.ShapeDtypeStruct((n // _JB, wb, n), jnp.float32),
        ],
        compiler_params=pltpu.CompilerParams(
            dimension_semantics=("arbitrary",),
        ),
    )(adj, elT, er, feat, bias, m8)


def _sc_layer2(pt, tbl, b2t, d2t):
    info = plsc.get_sparse_core_info()
    nc = info.num_cores
    nwk = nc * info.num_subcores
    mesh = plsc.VectorSubcoreMesh(core_axis_name="c", subcore_axis_name="s")
    nstrips = _N // 8
    npad = _NWP * 8

    @functools.partial(
        pl.kernel, mesh=mesh,
        compiler_params=pltpu.CompilerParams(needs_layout_passes=False),
        out_type=jax.ShapeDtypeStruct((_N, 48), jnp.float32),
        scratch_types=[
            pltpu.VMEM((4 * _NWP * 8,), jnp.float32),
            pltpu.VMEM((_NWP * 8,), jnp.float32),
            pltpu.VMEM((_NWP * 8,), jnp.float32),
            pltpu.VMEM((8, _NWP), jnp.float32),
            pltpu.VMEM((8, _NWP), jnp.float32),
            pltpu.VMEM((8, 48), jnp.float32),
            pltpu.SemaphoreType.DMA,
        ],
    )
    def k(pt_hbm, tbl_hbm, b2_hbm, d2_hbm, out_hbm,
          tblv, b2v, d2v, s0, s1, stage, sem):
        wid = lax.axis_index("s") * nc + lax.axis_index("c")
        pltpu.sync_copy(tbl_hbm, tblv)
        pltpu.sync_copy(b2_hbm, b2v)
        pltpu.sync_copy(d2_hbm, d2v)
        iot = lax.iota(jnp.int32, 16)
        zero = jnp.zeros((16,), jnp.float32)
        nt = (nstrips - 1 - wid) // nwk + 1

        def start(t, buf):
            g = wid + nwk * t
            pltpu.async_copy(pt_hbm.at[pl.ds(g * 8, 8), :], buf, sem)

        def wait(buf):
            pltpu.make_async_copy(pt_hbm.at[pl.ds(0, 8), :], buf, sem).wait()

        def process_strip(t, buf):
            g = wid + nwk * t
            dbase = g * 8
            for r in range(8):
                dst = dbase + r
                dvec = iot * 0 + dst
                bs = plsc.load_gather(b2v, [dvec])
                dsv = plsc.load_gather(d2v, [dvec])

                def scan_body(kk, acc):
                    wv = buf[r, pl.ds(kk * 16, 16)]

                    def dec(a):
                        an0, an1, ad = a
                        wi = wv.astype(jnp.int32)
                        idx0 = kk * 128 + iot * 8
                        for s in range(8):
                            m = ((wi >> s) & 1) == 1
                            idxs = idx0 + s
                            av = plsc.load_gather(tblv, [idxs], mask=m)
                            cv = plsc.load_gather(tblv, [idxs + npad], mask=m)
                            f0v = plsc.load_gather(tblv, [idxs + 2 * npad],
                                                   mask=m)
                            f1v = plsc.load_gather(tblv, [idxs + 3 * npad],
                                                   mask=m)
                            w = jnp.maximum(av * bs, cv * dsv)
                            w = jnp.where(m, w, 0.0)
                            an0 = an0 + w * f0v
                            an1 = an1 + w * f1v
                            ad = ad + w
                        return (an0, an1, ad)

                    return lax.cond(jnp.any(wv != 0.0), dec, lambda a: a, acc)

                an0, an1, ad = lax.fori_loop(0, _NWP // 16, scan_body,
                                             (zero, zero, zero))
                stage[r, pl.ds(0, 16)] = an0
                stage[r, pl.ds(16, 16)] = an1
                stage[r, pl.ds(32, 16)] = ad
            pltpu.sync_copy(stage, out_hbm.at[pl.ds(dbase, 8), :])

        @pl.when(nt > 0)
        def _prime():
            start(0, s0)

        def pair_body(p, carry):
            for b, buf, obuf in ((0, s0, s1), (1, s1, s0)):
                t = 2 * p + b

                @pl.when(t < nt)
                def _do():
                    @pl.when(t + 1 < nt)
                    def _next():
                        start(t + 1, obuf)
                    wait(buf)
                    process_strip(t, buf)
            return carry

        lax.fori_loop(0, (nt + 1) // 2, pair_body, jnp.int32(0))

    return k(pt, tbl, b2t, d2t)


def _epi_body(stg_ref, b_ref, out_ref):
    stg = stg_ref[...]
    n0 = jnp.sum(stg[:, 0:16], axis=1, keepdims=True)
    n1 = jnp.sum(stg[:, 16:32], axis=1, keepdims=True)
    den = jnp.sum(stg[:, 32:48], axis=1, keepdims=True)
    safe = jnp.where(den > 0, den, 1.0)
    lg = jnp.concatenate([n0, n1], axis=1) / safe + b_ref[...]
    m = jnp.max(lg, axis=1, keepdims=True)
    lse = m + jnp.log(jnp.sum(jnp.exp(lg - m), axis=1, keepdims=True))
    out_ref[...] = lg - lse


def _epilogue(stg, b2r):
    n = stg.shape[0]
    return pl.pallas_call(
        _epi_body,
        grid=(n // _RB,),
        in_specs=[
            pl.BlockSpec((_RB, 48), lambda i: (i, 0)),
            pl.BlockSpec((1, 2), lambda i: (0, 0)),
        ],
        out_specs=pl.BlockSpec((_RB, 2), lambda i: (i, 0)),
        out_shape=jax.ShapeDtypeStruct((n, 2), jnp.float32),
    )(stg, b2r)


def kernel(x, adj, W1, al1, ar1, b1, W2, al2, ar2, b2):
    eye = jnp.eye(8, dtype=jnp.float32)
    AL1 = (al1.reshape(8, 16)[:, :, None] * eye[:, None, :]).reshape(128, 8)
    AR1 = (ar1.reshape(8, 16)[:, :, None] * eye[:, None, :]).reshape(128, 8)
    AL2 = jnp.pad(al2.reshape(2, 1), ((0, 0), (0, 7)))
    AR2 = jnp.pad(ar2.reshape(2, 1), ((0, 0), (0, 7)))
    b1r = b1.reshape(1, 128)
    b2r = b2.reshape(1, 2)
    wb = _JB // 8
    rows = jnp.arange(_JB)
    m8 = jnp.where(rows[None, :] // 8 == jnp.arange(wb)[:, None],
                   2.0 ** (rows[None, :] % 8), 0.0).astype(jnp.bfloat16)

    f1, el1, er1 = _transform(x, W1, AL1, AR1, H=8, D=16)
    h1, P = _message_pass1(adj, el1.T, er1, f1, b1r, m8, H=8, D=16)
    P = P.reshape(_NW, _N)
    pt = jnp.pad(P.T, ((0, 0), (0, _NWP - _NW)))
    _, el2, er2, t8 = _transform(h1, W2, AL2, AR2, H=1, D=2, tables=True)
    zpad = jnp.zeros((_NWP * 8 - _N,), jnp.float32)
    tbl = jnp.concatenate([t8[:, 0], zpad, t8[:, 1], zpad,
                           t8[:, 2], zpad, t8[:, 3], zpad])
    stg = _sc_layer2(pt, tbl,
                     jnp.concatenate([t8[:, 4], zpad]),
                     jnp.concatenate([t8[:, 5], zpad]))
    return _epilogue(stg, b2r)


# speedup vs baseline: 1.5718x; 1.4288x over previous
import functools

import jax
import jax.numpy as jnp
from jax import lax
from jax.experimental import pallas as pl
from jax.experimental.pallas import tpu as pltpu
from jax.experimental.pallas import tpu_sc as plsc

_N = 10000
_JB = 200
_RB = 1000
_NW = _N // 8
_NWP = 1280


def _tf_body(x_ref, w_ref, al_ref, ar_ref, f_ref, el_ref, er_ref, *rest,
             H, D, tables):
    f = jnp.dot(x_ref[...], w_ref[...], preferred_element_type=jnp.float32)
    el = jnp.dot(f, al_ref[...], preferred_element_type=jnp.float32)
    er = jnp.dot(f, ar_ref[...], preferred_element_type=jnp.float32)
    el_ref[...] = el
    er_ref[...] = er
    ones = jnp.ones((f.shape[0], 1), dtype=jnp.bfloat16)
    fb = f.astype(jnp.bfloat16)
    pieces = []
    for h in range(H):
        pieces.append(fb[:, h * D:(h + 1) * D])
        pieces.append(ones)
    f_ref[...] = jnp.concatenate(pieces, axis=1)
    if tables:
        t_ref = rest[0]
        e0 = el[:, 0:1]
        r0 = er[:, 0:1]
        t_ref[...] = jnp.concatenate([
            jnp.exp(e0), jnp.exp(0.2 * e0),
            f[:, 0:1], f[:, 1:2],
            jnp.exp(r0), jnp.exp(0.2 * r0),
            jnp.zeros((f.shape[0], 2), jnp.float32),
        ], axis=1)


def _transform(h, W, AL, AR, H, D, tables=False):
    n = h.shape[0]
    kin = h.shape[1]
    kout = W.shape[1]
    hh = AL.shape[1]
    body = functools.partial(_tf_body, H=H, D=D, tables=tables)
    out_specs = [
        pl.BlockSpec((_RB, H * (D + 1)), lambda i: (i, 0)),
        pl.BlockSpec((_RB, hh), lambda i: (i, 0)),
        pl.BlockSpec((_RB, hh), lambda i: (i, 0)),
    ]
    out_shape = [
        jax.ShapeDtypeStruct((n, H * (D + 1)), jnp.bfloat16),
        jax.ShapeDtypeStruct((n, hh), jnp.float32),
        jax.ShapeDtypeStruct((n, hh), jnp.float32),
    ]
    if tables:
        out_specs.append(pl.BlockSpec((_RB, 8), lambda i: (i, 0)))
        out_shape.append(jax.ShapeDtypeStruct((n, 8), jnp.float32))
    return pl.pallas_call(
        body,
        grid=(n // _RB,),
        in_specs=[
            pl.BlockSpec((_RB, kin), lambda i: (i, 0)),
            pl.BlockSpec((kin, kout), lambda i: (0, 0)),
            pl.BlockSpec((kout, hh), lambda i: (0, 0)),
            pl.BlockSpec((kout, hh), lambda i: (0, 0)),
        ],
        out_specs=out_specs,
        out_shape=out_shape,
    )(h, W, AL, AR)


def _mp1_body(adj_ref, elT_ref, er_ref, feat_ref, bias_ref, m8_ref,
              out_ref, p_ref, *, H, D):
    adjb = adj_ref[...].astype(jnp.bfloat16)
    p_ref[0, :, :] = jnp.dot(m8_ref[...], adjb, preferred_element_type=jnp.float32)
    elT = elT_ref[...]
    er = er_ref[...]
    A = jnp.exp(elT).astype(jnp.bfloat16)
    C = jnp.exp(0.2 * elT).astype(jnp.bfloat16)
    B = jnp.exp(er).astype(jnp.bfloat16)
    Dd = jnp.exp(0.2 * er).astype(jnp.bfloat16)
    fbf = feat_ref[...]
    nums = []
    for h in range(H):
        a = A[h:h + 1, :]
        c = C[h:h + 1, :]
        b = B[:, h:h + 1]
        d = Dd[:, h:h + 1]
        w = adjb * jnp.maximum(b * a, d * c)
        nd = jnp.dot(w, fbf[:, h * (D + 1):(h + 1) * (D + 1)],
                     preferred_element_type=jnp.float32)
        den = nd[:, D:D + 1]
        safe = jnp.where(den > 0, den, 1.0)
        nums.append(nd[:, 0:D] / safe)
    res = jnp.concatenate(nums, axis=1) + bias_ref[...]
    out_ref[...] = jnp.where(res > 0, res,
                             jnp.exp(jnp.minimum(res, 0.0)) - 1.0)


def _message_pass1(adj, elT, er, feat, bias, m8, H, D):
    n = adj.shape[0]
    hd = feat.shape[1]
    wb = _JB // 8
    body = functools.partial(_mp1_body, H=H, D=D)
    return pl.pallas_call(
        body,
        grid=(n // _JB,),
        in_specs=[
            pl.BlockSpec((_JB, n), lambda j: (j, 0)),
            pl.BlockSpec((8, n), lambda j: (0, 0)),
            pl.BlockSpec((_JB, 8), lambda j: (j, 0)),
            pl.BlockSpec((n, hd), lambda j: (0, 0)),
            pl.BlockSpec((1, H * D), lambda j: (0, 0)),
            pl.BlockSpec((wb, _JB), lambda j: (0, 0)),
        ],
        out_specs=[
            pl.BlockSpec((_JB, H * D), lambda j: (j, 0)),
            pl.BlockSpec((1, wb, n), lambda j: (j, 0, 0)),
        ],
        out_shape=[
            jax.ShapeDtypeStruct((n, H * D), jnp.float32),
            jax.ShapeDtypeStruct((n // _JB, wb, n), jnp.float32),
        ],
        compiler_params=pltpu.CompilerParams(
            dimension_semantics=("arbitrary",),
        ),
    )(adj, elT, er, feat, bias, m8)


def _sc_layer2(pt, tbl, b2t, d2t):
    info = plsc.get_sparse_core_info()
    nc = info.num_cores
    nwk = nc * info.num_subcores
    mesh = plsc.VectorSubcoreMesh(core_axis_name="c", subcore_axis_name="s")
    nstrips = _N // 8
    npad = _NWP * 8

    @functools.partial(
        pl.kernel, mesh=mesh,
        compiler_params=pltpu.CompilerParams(needs_layout_passes=False),
        out_type=jax.ShapeDtypeStruct((_N, 48), jnp.float32),
        scratch_types=[
            pltpu.VMEM((4 * _NWP * 8,), jnp.float32),
            pltpu.VMEM((_NWP * 8,), jnp.float32),
            pltpu.VMEM((_NWP * 8,), jnp.float32),
            pltpu.VMEM((1, _N), jnp.float32),
            pltpu.VMEM((1, _N), jnp.float32),
            pltpu.VMEM((8, 48), jnp.float32),
            pltpu.SemaphoreType.DMA,
        ],
    )
    def k(p_hbm, tbl_hbm, b2_hbm, d2_hbm, out_hbm,
          tblv, b2v, d2v, s0, s1, stage, sem):
        wid = lax.axis_index("s") * nc + lax.axis_index("c")
        pltpu.sync_copy(tbl_hbm, tblv)
        pltpu.sync_copy(b2_hbm, b2v)
        pltpu.sync_copy(d2_hbm, d2v)
        iot = lax.iota(jnp.int32, 16)
        zero = jnp.zeros((16,), jnp.float32)
        nt = (nstrips - 1 - wid) // nwk + 1

        def start(t, buf):
            g = wid + nwk * t
            pltpu.async_copy(p_hbm.at[pl.ds(g, 1), :], buf, sem)

        def wait(buf):
            pltpu.make_async_copy(p_hbm.at[pl.ds(0, 1), :], buf, sem).wait()

        def process_strip(t, buf):
            g = wid + nwk * t
            dbase = g * 8
            bsp = []
            dsp = []
            for s in range(8):
                dvec = iot * 0 + (dbase + s)
                bsp.append(plsc.load_gather(b2v, [dvec]))
                dsp.append(plsc.load_gather(d2v, [dvec]))

            def scan_body(kk, acc):
                base = kk * 80
                vs = [buf[0, pl.ds(base + 16 * i, 16)] for i in range(5)]
                tot = vs[0] + vs[1] + vs[2] + vs[3] + vs[4]

                def dec(a):
                    for i in range(5):
                        wv = vs[i]

                        def dec1(aa, wv=wv, i=i):
                            aa = list(aa)
                            wi = wv.astype(jnp.int32)
                            off = base + 16 * i
                            av = tblv[pl.ds(off, 16)]
                            cv = tblv[pl.ds(npad + off, 16)]
                            f0v = tblv[pl.ds(2 * npad + off, 16)]
                            f1v = tblv[pl.ds(3 * npad + off, 16)]
                            for s in range(8):
                                m = ((wi >> s) & 1) == 1
                                w = jnp.maximum(av * bsp[s], cv * dsp[s])
                                w = jnp.where(m, w, 0.0)
                                aa[3 * s] = aa[3 * s] + w * f0v
                                aa[3 * s + 1] = aa[3 * s + 1] + w * f1v
                                aa[3 * s + 2] = aa[3 * s + 2] + w
                            return tuple(aa)

                        a = lax.cond(jnp.any(wv != 0.0), dec1,
                                     lambda aa: aa, a)
                    return a

                return lax.cond(jnp.any(tot != 0.0), dec, lambda a: a, acc)

            acc0 = tuple(zero for _ in range(24))
            acc = lax.fori_loop(0, _N // 80, scan_body, acc0)
            for s in range(8):
                stage[s, pl.ds(0, 16)] = acc[3 * s]
                stage[s, pl.ds(16, 16)] = acc[3 * s + 1]
                stage[s, pl.ds(32, 16)] = acc[3 * s + 2]
            pltpu.sync_copy(stage, out_hbm.at[pl.ds(dbase, 8), :])

        @pl.when(nt > 0)
        def _prime():
            start(0, s0)

        def pair_body(p, carry):
            for b, buf, obuf in ((0, s0, s1), (1, s1, s0)):
                t = 2 * p + b

                @pl.when(t < nt)
                def _do():
                    @pl.when(t + 1 < nt)
                    def _next():
                        start(t + 1, obuf)
                    wait(buf)
                    process_strip(t, buf)
            return carry

        lax.fori_loop(0, (nt + 1) // 2, pair_body, jnp.int32(0))

    return k(pt, tbl, b2t, d2t)


def _epi_body(stg_ref, b_ref, out_ref):
    stg = stg_ref[...]
    n0 = jnp.sum(stg[:, 0:16], axis=1, keepdims=True)
    n1 = jnp.sum(stg[:, 16:32], axis=1, keepdims=True)
    den = jnp.sum(stg[:, 32:48], axis=1, keepdims=True)
    safe = jnp.where(den > 0, den, 1.0)
    lg = jnp.concatenate([n0, n1], axis=1) / safe + b_ref[...]
    m = jnp.max(lg, axis=1, keepdims=True)
    lse = m + jnp.log(jnp.sum(jnp.exp(lg - m), axis=1, keepdims=True))
    out_ref[...] = lg - lse


def _epilogue(stg, b2r):
    n = stg.shape[0]
    return pl.pallas_call(
        _epi_body,
        grid=(n // _RB,),
        in_specs=[
            pl.BlockSpec((_RB, 48), lambda i: (i, 0)),
            pl.BlockSpec((1, 2), lambda i: (0, 0)),
        ],
        out_specs=pl.BlockSpec((_RB, 2), lambda i: (i, 0)),
        out_shape=jax.ShapeDtypeStruct((n, 2), jnp.float32),
    )(stg, b2r)


def kernel(x, adj, W1, al1, ar1, b1, W2, al2, ar2, b2):
    eye = jnp.eye(8, dtype=jnp.float32)
    AL1 = (al1.reshape(8, 16)[:, :, None] * eye[:, None, :]).reshape(128, 8)
    AR1 = (ar1.reshape(8, 16)[:, :, None] * eye[:, None, :]).reshape(128, 8)
    AL2 = jnp.pad(al2.reshape(2, 1), ((0, 0), (0, 7)))
    AR2 = jnp.pad(ar2.reshape(2, 1), ((0, 0), (0, 7)))
    b1r = b1.reshape(1, 128)
    b2r = b2.reshape(1, 2)
    wb = _JB // 8
    rows = jnp.arange(_JB)
    m8 = jnp.where(rows[None, :] // 8 == jnp.arange(wb)[:, None],
                   2.0 ** (rows[None, :] % 8), 0.0).astype(jnp.bfloat16)

    f1, el1, er1 = _transform(x, W1, AL1, AR1, H=8, D=16)
    h1, P = _message_pass1(adj, el1.T, er1, f1, b1r, m8, H=8, D=16)
    P = P.reshape(_NW, _N)
    _, el2, er2, t8 = _transform(h1, W2, AL2, AR2, H=1, D=2, tables=True)
    zpad = jnp.zeros((_NWP * 8 - _N,), jnp.float32)
    tbl = jnp.concatenate([t8[:, 0], zpad, t8[:, 1], zpad,
                           t8[:, 2], zpad, t8[:, 3], zpad])
    stg = _sc_layer2(P, tbl,
                     jnp.concatenate([t8[:, 4], zpad]),
                     jnp.concatenate([t8[:, 5], zpad]))
    return _epilogue(stg, b2r)
